# Initial kernel scaffold; baseline (speedup 1.0000x reference)
import jax
import jax.numpy as jnp
from jax.experimental import pallas as pl


def kernel(x, edge_index, edge_weights, feature_mask, conv_weights, conv_biases, bn_gamma, bn_beta, fc_w, fc_b):
    raise NotImplementedError("write your pallas kernel here")


# speedup vs baseline: 1.0455x; 1.0455x over previous
import functools

import jax
import jax.numpy as jnp
from jax.experimental import pallas as pl
from jax.experimental.pallas import tpu as pltpu

_N_NODES = 2000
_N_EDGES = 6000
_BATCH = 4
_DIMS = [8, 32, 16, 8, 2]
_HOPS = [3, 3, 3]
_SLOPE = 0.01
_BN_EPS = 1e-5

_FC_IN = _DIMS[-2] * _N_NODES
_FC_OUT = _DIMS[-1] * _N_NODES

_KT = 3200
_NT = 512
_FC_OUT_PAD = 4096


def _fc_body(x_ref, w_ref, b_ref, m_ref, o_ref):
    n = pl.program_id(0)
    k = pl.program_id(1)
    nk = pl.num_programs(1)

    @pl.when(k == 0)
    def _init():
        o_ref[...] = jnp.zeros_like(o_ref)

    o_ref[...] += jnp.dot(x_ref[...], w_ref[...],
                          preferred_element_type=jnp.float32)

    @pl.when(k == nk - 1)
    def _fini():
        b = b_ref[0, pl.ds(n * _NT, _NT)]
        m = m_ref[0, pl.ds(n * _NT, _NT)]
        o_ref[...] = (o_ref[...] + b[None, :]) * m[None, :]


@jax.jit
def _fc_pallas(x2d, fc_w, fc_b, mask_flat):
    pad = _FC_OUT_PAD - _FC_OUT
    b_pad = jnp.pad(fc_b, (0, pad)).reshape(1, _FC_OUT_PAD)
    m_pad = jnp.pad(mask_flat, (0, pad)).reshape(1, _FC_OUT_PAD)
    grid = (_FC_OUT_PAD // _NT, _FC_IN // _KT)
    y = pl.pallas_call(
        _fc_body,
        grid=grid,
        in_specs=[
            pl.BlockSpec((_BATCH, _KT), lambda n, k: (0, k)),
            pl.BlockSpec((_KT, _NT), lambda n, k: (k, n)),
            pl.BlockSpec((1, _FC_OUT_PAD), lambda n, k: (0, 0)),
            pl.BlockSpec((1, _FC_OUT_PAD), lambda n, k: (0, 0)),
        ],
        out_specs=pl.BlockSpec((_BATCH, _NT), lambda n, k: (0, n)),
        out_shape=jax.ShapeDtypeStruct((_BATCH, _FC_OUT_PAD), jnp.float32),
        compiler_params=pltpu.CompilerParams(
            dimension_semantics=("parallel", "arbitrary"),
        ),
    )(x2d, fc_w, b_pad, m_pad)
    return y[:, :_FC_OUT]


def _graph_phase(x, src, dst, ew, conv_weights, conv_biases, bn_gamma, bn_beta):
    deg = jnp.zeros((_N_NODES,), jnp.float32).at[dst].add(ew)
    dinv = jnp.where(deg > 0, 1.0 / jnp.sqrt(deg), 0.0)
    norm = dinv[src] * ew * dinv[dst]

    out = x
    for layer in range(len(_HOPS)):
        h = out
        acc = h @ conv_weights[layer][0]
        for w_hop in conv_weights[layer][1:]:
            msgs = h[:, src, :] * norm[None, :, None]
            h = jnp.zeros_like(h).at[:, dst, :].add(msgs)
            acc = acc + h @ w_hop
        out = acc + conv_biases[layer]
        mu = jnp.mean(out, axis=(0, 2), keepdims=True)
        var = jnp.var(out, axis=(0, 2), keepdims=True)
        out = (out - mu) / jnp.sqrt(var + _BN_EPS)
        out = out * bn_gamma[layer][None, :, None] + bn_beta[layer][None, :, None]
        out = jnp.where(out >= 0, out, _SLOPE * out)
    return out


def kernel(x, edge_index, edge_weights, feature_mask, conv_weights,
           conv_biases, bn_gamma, bn_beta, fc_w, fc_b):
    src, dst = edge_index[0], edge_index[1]
    out = _graph_phase(x, src, dst, edge_weights, conv_weights, conv_biases,
                       bn_gamma, bn_beta)
    x2d = out.reshape(_BATCH, _FC_IN)
    y = _fc_pallas(x2d, fc_w, fc_b, feature_mask.reshape(-1))
    return y.reshape(_BATCH, _N_NODES, _DIMS[-1])
